# Initial kernel scaffold; baseline (speedup 1.0000x reference)
import jax
import jax.numpy as jnp
from jax.experimental import pallas as pl


def kernel(features, AM, W, b, att_src, att_dst):
    raise NotImplementedError("write your pallas kernel here")


# speedup vs baseline: 9004.1992x; 9004.1992x over previous
import jax
import jax.numpy as jnp
from jax.experimental import pallas as pl
from jax.experimental.pallas import tpu as pltpu

_N = 2048
_IN_D = 128
_OUT_D = 16
_H = 4
_BI = 256
_GRID = _N // _BI
_NEG_SLOPE = 0.2


def _gat_body(feat_ref, am_ref, w_ref, b_ref, asrc_ref, adst_ref,
              out_ref,
              wh_scr, acs_scr, ea1_scr, ea2_scr,
              acdT_scr, ed1_scr, ed2_scr):
    i = pl.program_id(0)

    @pl.when(i == 0)
    def _prologue():
        wh = jax.lax.dot_general(
            feat_ref[...], w_ref[...], (((1,), (1,)), ((), ())),
            preferred_element_type=jnp.float32) + b_ref[...]
        wh_scr[...] = wh
        acs = jax.lax.dot_general(
            wh, asrc_ref[...], (((1,), (1,)), ((), ())),
            preferred_element_type=jnp.float32)
        acd = jax.lax.dot_general(
            wh, adst_ref[...], (((1,), (1,)), ((), ())),
            preferred_element_type=jnp.float32)
        acdT = jax.lax.dot_general(
            adst_ref[...], wh, (((1,), (1,)), ((), ())),
            preferred_element_type=jnp.float32)
        m = (jnp.max(acs, axis=0, keepdims=True)
             + jnp.max(acd, axis=0, keepdims=True))
        acs_scr[...] = acs
        acdT_scr[...] = acdT
        ea1_scr[...] = jnp.exp(acs - m)
        ea2_scr[...] = jnp.exp(_NEG_SLOPE * acs - m)
        ed1_scr[...] = jnp.exp(acdT)
        ed2_scr[...] = jnp.exp(_NEG_SLOPE * acdT)
        out_ref[...] = jnp.zeros_like(out_ref)

    rows = pl.ds(i * _BI, _BI)
    mask = am_ref[...] != 0
    acs_blk = acs_scr[rows, :]
    ea1_blk = ea1_scr[rows, :]
    ea2_blk = ea2_scr[rows, :]
    wh_blk = wh_scr[rows, :]

    for h in range(_H):
        cond = (acs_blk[:, h:h + 1] + acdT_scr[h:h + 1, :]) >= 0.0
        u = ea1_blk[:, h:h + 1] * ed1_scr[h:h + 1, :]
        v = ea2_blk[:, h:h + 1] * ed2_scr[h:h + 1, :]
        e = jnp.where(mask, jnp.where(cond, u, v), 0.0)
        s = jnp.sum(e, axis=1, keepdims=True)
        whp = wh_blk * (1.0 / (s + 1e-16))
        out_ref[h] += jax.lax.dot_general(
            e, whp, (((0,), (0,)), ((), ())),
            preferred_element_type=jnp.float32)


def kernel(features, AM, W, b, att_src, att_dst):
    out = pl.pallas_call(
        _gat_body,
        grid=(_GRID,),
        in_specs=[
            pl.BlockSpec((_N, _IN_D), lambda i: (0, 0)),
            pl.BlockSpec((_BI, _N), lambda i: (i, 0)),
            pl.BlockSpec((_OUT_D, _IN_D), lambda i: (0, 0)),
            pl.BlockSpec((1, _OUT_D), lambda i: (0, 0)),
            pl.BlockSpec((_H, _OUT_D), lambda i: (0, 0)),
            pl.BlockSpec((_H, _OUT_D), lambda i: (0, 0)),
        ],
        out_specs=pl.BlockSpec((_H, _N, _OUT_D), lambda i: (0, 0, 0)),
        out_shape=jax.ShapeDtypeStruct((_H, _N, _OUT_D), jnp.float32),
        scratch_shapes=[
            pltpu.VMEM((_N, _OUT_D), jnp.float32),
            pltpu.VMEM((_N, _H), jnp.float32),
            pltpu.VMEM((_N, _H), jnp.float32),
            pltpu.VMEM((_N, _H), jnp.float32),
            pltpu.VMEM((_H, _N), jnp.float32),
            pltpu.VMEM((_H, _N), jnp.float32),
            pltpu.VMEM((_H, _N), jnp.float32),
        ],
    )(features, AM, W, jnp.reshape(b, (1, _OUT_D)), att_src, att_dst)
    return jnp.transpose(out, (1, 0, 2)).reshape(_N, _H * _OUT_D)


# speedup vs baseline: 10962.6820x; 1.2175x over previous
import jax
import jax.numpy as jnp
from jax.experimental import pallas as pl
from jax.experimental.pallas import tpu as pltpu

_N = 2048
_IN_D = 128
_OUT_D = 16
_H = 4
_BI = 256
_GRID = _N // _BI
_NEG_SLOPE = 0.2


def _gat_body(feat_ref, am_ref, w_ref, b_ref, asrc_ref, adst_ref,
              out_ref,
              wh_scr, ea1_scr, ea2_scr, ed1_scr, ed2_scr, accT_scr):
    i = pl.program_id(0)

    @pl.when(i == 0)
    def _prologue():
        wh = jax.lax.dot_general(
            feat_ref[...], w_ref[...], (((1,), (1,)), ((), ())),
            preferred_element_type=jnp.float32) + b_ref[...]
        wh_scr[...] = wh
        acs = jax.lax.dot_general(
            wh, asrc_ref[...], (((1,), (1,)), ((), ())),
            preferred_element_type=jnp.float32)
        acd = jax.lax.dot_general(
            wh, adst_ref[...], (((1,), (1,)), ((), ())),
            preferred_element_type=jnp.float32)
        acdT = jax.lax.dot_general(
            adst_ref[...], wh, (((1,), (1,)), ((), ())),
            preferred_element_type=jnp.float32)
        m = (jnp.max(acs, axis=0, keepdims=True)
             + jnp.max(acd, axis=0, keepdims=True))
        ea1_scr[...] = jnp.exp(acs - m)
        ea2_scr[...] = jnp.exp(_NEG_SLOPE * acs - m)
        ed1_scr[...] = jnp.exp(acdT)
        ed2_scr[...] = jnp.exp(_NEG_SLOPE * acdT)
        accT_scr[...] = jnp.zeros_like(accT_scr)

    rows = pl.ds(i * _BI, _BI)
    maskb = am_ref[...].astype(jnp.bfloat16)
    ones_col = jnp.ones((_N, 8), dtype=jnp.bfloat16)
    ea1_blk = ea1_scr[rows, :]
    ea2_blk = ea2_scr[rows, :]
    wh_blk = wh_scr[rows, :]

    for h in range(_H):
        u = ea1_blk[:, h:h + 1] * ed1_scr[h:h + 1, :]
        v = ea2_blk[:, h:h + 1] * ed2_scr[h:h + 1, :]
        e = jnp.maximum(u, v).astype(jnp.bfloat16) * maskb
        s = jax.lax.dot_general(
            e, ones_col, (((1,), (0,)), ((), ())),
            preferred_element_type=jnp.float32)[:, 0:1]
        whp = (wh_blk * (1.0 / (s + 1e-16))).astype(jnp.bfloat16)
        accT_scr[h] += jax.lax.dot_general(
            whp, e, (((0,), (0,)), ((), ())),
            preferred_element_type=jnp.float32)

    @pl.when(i == _GRID - 1)
    def _epilogue():
        for h in range(_H):
            out_ref[h] = accT_scr[h].T


def kernel(features, AM, W, b, att_src, att_dst):
    out = pl.pallas_call(
        _gat_body,
        grid=(_GRID,),
        in_specs=[
            pl.BlockSpec((_N, _IN_D), lambda i: (0, 0)),
            pl.BlockSpec((_BI, _N), lambda i: (i, 0)),
            pl.BlockSpec((_OUT_D, _IN_D), lambda i: (0, 0)),
            pl.BlockSpec((1, _OUT_D), lambda i: (0, 0)),
            pl.BlockSpec((_H, _OUT_D), lambda i: (0, 0)),
            pl.BlockSpec((_H, _OUT_D), lambda i: (0, 0)),
        ],
        out_specs=pl.BlockSpec((_H, _N, _OUT_D), lambda i: (0, 0, 0)),
        out_shape=jax.ShapeDtypeStruct((_H, _N, _OUT_D), jnp.float32),
        scratch_shapes=[
            pltpu.VMEM((_N, _OUT_D), jnp.float32),
            pltpu.VMEM((_N, _H), jnp.float32),
            pltpu.VMEM((_N, _H), jnp.float32),
            pltpu.VMEM((_H, _N), jnp.float32),
            pltpu.VMEM((_H, _N), jnp.float32),
            pltpu.VMEM((_H, _OUT_D, _N), jnp.float32),
        ],
    )(features, AM, W, jnp.reshape(b, (1, _OUT_D)), att_src, att_dst)
    return jnp.transpose(out, (1, 0, 2)).reshape(_N, _H * _OUT_D)


# speedup vs baseline: 11818.5628x; 1.0781x over previous
import jax
import jax.numpy as jnp
from jax.experimental import pallas as pl
from jax.experimental.pallas import tpu as pltpu

_N = 2048
_IN_D = 128
_OUT_D = 16
_H = 4
_BI = 256
_GRID = _N // _BI
_NEG_SLOPE = 0.2


def _tables_body(feat_ref, w_ref, b_ref, asrc_ref, adst_ref,
                 wh_ref, ea1_ref, ea2_ref, ed1_ref, ed2_ref):
    wh = jax.lax.dot_general(
        feat_ref[...], w_ref[...], (((1,), (1,)), ((), ())),
        preferred_element_type=jnp.float32) + b_ref[...]
    wh_ref[...] = wh
    acs = jax.lax.dot_general(
        wh, asrc_ref[...], (((1,), (1,)), ((), ())),
        preferred_element_type=jnp.float32)
    acd = jax.lax.dot_general(
        wh, adst_ref[...], (((1,), (1,)), ((), ())),
        preferred_element_type=jnp.float32)
    acdT = jax.lax.dot_general(
        adst_ref[...], wh, (((1,), (1,)), ((), ())),
        preferred_element_type=jnp.float32)
    m = (jnp.max(acs, axis=0, keepdims=True)
         + jnp.max(acd, axis=0, keepdims=True))
    ea1_ref[...] = jnp.exp(acs - m)
    ea2_ref[...] = jnp.exp(_NEG_SLOPE * acs - m)
    ed1_ref[...] = jnp.exp(acdT)
    ed2_ref[...] = jnp.exp(_NEG_SLOPE * acdT)


def _gat_body(am_ref, wh_ref, ea1_ref, ea2_ref, ed1_ref, ed2_ref,
              outT_ref):
    i = pl.program_id(0)

    @pl.when(i == 0)
    def _init():
        outT_ref[...] = jnp.zeros_like(outT_ref)

    rows = pl.ds(i * _BI, _BI)
    maskb = am_ref[...].astype(jnp.bfloat16)
    ones_col = jnp.ones((_N, 8), dtype=jnp.bfloat16)
    ea1_blk = ea1_ref[rows, :]
    ea2_blk = ea2_ref[rows, :]
    wh_blk = wh_ref[rows, :]

    for h in range(_H):
        u = ea1_blk[:, h:h + 1] * ed1_ref[h:h + 1, :]
        v = ea2_blk[:, h:h + 1] * ed2_ref[h:h + 1, :]
        e = jnp.maximum(u, v).astype(jnp.bfloat16) * maskb
        s = jax.lax.dot_general(
            e, ones_col, (((1,), (0,)), ((), ())),
            preferred_element_type=jnp.float32)[:, 0:1]
        whp = (wh_blk * (1.0 / (s + 1e-16))).astype(jnp.bfloat16)
        outT_ref[h] += jax.lax.dot_general(
            whp, e, (((0,), (0,)), ((), ())),
            preferred_element_type=jnp.float32)


def kernel(features, AM, W, b, att_src, att_dst):
    wh, ea1, ea2, ed1, ed2 = pl.pallas_call(
        _tables_body,
        in_specs=[
            pl.BlockSpec((_N, _IN_D), lambda: (0, 0)),
            pl.BlockSpec((_OUT_D, _IN_D), lambda: (0, 0)),
            pl.BlockSpec((1, _OUT_D), lambda: (0, 0)),
            pl.BlockSpec((_H, _OUT_D), lambda: (0, 0)),
            pl.BlockSpec((_H, _OUT_D), lambda: (0, 0)),
        ],
        out_specs=[
            pl.BlockSpec((_N, _OUT_D), lambda: (0, 0)),
            pl.BlockSpec((_N, _H), lambda: (0, 0)),
            pl.BlockSpec((_N, _H), lambda: (0, 0)),
            pl.BlockSpec((_H, _N), lambda: (0, 0)),
            pl.BlockSpec((_H, _N), lambda: (0, 0)),
        ],
        out_shape=[
            jax.ShapeDtypeStruct((_N, _OUT_D), jnp.float32),
            jax.ShapeDtypeStruct((_N, _H), jnp.float32),
            jax.ShapeDtypeStruct((_N, _H), jnp.float32),
            jax.ShapeDtypeStruct((_H, _N), jnp.float32),
            jax.ShapeDtypeStruct((_H, _N), jnp.float32),
        ],
    )(features, W, jnp.reshape(b, (1, _OUT_D)), att_src, att_dst)

    outT = pl.pallas_call(
        _gat_body,
        grid=(_GRID,),
        in_specs=[
            pl.BlockSpec((_BI, _N), lambda i: (i, 0)),
            pl.BlockSpec((_N, _OUT_D), lambda i: (0, 0)),
            pl.BlockSpec((_N, _H), lambda i: (0, 0)),
            pl.BlockSpec((_N, _H), lambda i: (0, 0)),
            pl.BlockSpec((_H, _N), lambda i: (0, 0)),
            pl.BlockSpec((_H, _N), lambda i: (0, 0)),
        ],
        out_specs=pl.BlockSpec((_H, _OUT_D, _N), lambda i: (0, 0, 0)),
        out_shape=jax.ShapeDtypeStruct((_H, _OUT_D, _N), jnp.float32),
    )(AM, wh, ea1, ea2, ed1, ed2)

    return jnp.transpose(outT, (2, 0, 1)).reshape(_N, _H * _OUT_D)


# speedup vs baseline: 19149.9960x; 1.6203x over previous
import jax
import jax.numpy as jnp
from jax.experimental import pallas as pl
from jax.experimental.pallas import tpu as pltpu

_N = 2048
_IN_D = 128
_OUT_D = 16
_H = 4
_BI = 1024
_GRID = _N // _BI
_NEG_SLOPE = 0.2


def _gat_body(feat_ref, am_ref, w_ref, b_ref, asrc_ref, adst_ref,
              outT_ref,
              wh_scr, ea1_scr, ea2_scr, ed1_scr, ed2_scr):
    g = pl.program_id(0)

    @pl.when(g == 0)
    def _tables():
        wh = jax.lax.dot_general(
            feat_ref[...], w_ref[...], (((1,), (1,)), ((), ())),
            preferred_element_type=jnp.float32) + b_ref[...]
        wh_scr[...] = wh
        acs = jax.lax.dot_general(
            wh, asrc_ref[...], (((1,), (1,)), ((), ())),
            preferred_element_type=jnp.float32)
        acd = jax.lax.dot_general(
            wh, adst_ref[...], (((1,), (1,)), ((), ())),
            preferred_element_type=jnp.float32)
        acdT = jax.lax.dot_general(
            adst_ref[...], wh, (((1,), (1,)), ((), ())),
            preferred_element_type=jnp.float32)
        m = (jnp.max(acs, axis=0, keepdims=True)
             + jnp.max(acd, axis=0, keepdims=True))
        ea1_scr[...] = jnp.exp(acs - m).astype(jnp.bfloat16)
        ea2_scr[...] = jnp.exp(_NEG_SLOPE * acs - m).astype(jnp.bfloat16)
        ed1_scr[...] = jnp.exp(acdT).astype(jnp.bfloat16)
        ed2_scr[...] = jnp.exp(_NEG_SLOPE * acdT).astype(jnp.bfloat16)
        outT_ref[...] = jnp.zeros_like(outT_ref)

    @pl.when(g > 0)
    def _block():
        i = g - 1
        rows = pl.ds(i * _BI, _BI)
        maskb = am_ref[...].astype(jnp.bfloat16)
        ones_col = jnp.ones((_N, 8), dtype=jnp.bfloat16)
        ea1_blk = ea1_scr[rows, :]
        ea2_blk = ea2_scr[rows, :]
        wh_blk = wh_scr[rows, :]

        for h in range(_H):
            u = ea1_blk[:, h:h + 1] * ed1_scr[h:h + 1, :]
            v = ea2_blk[:, h:h + 1] * ed2_scr[h:h + 1, :]
            e = jnp.maximum(u, v) * maskb
            if h % 2 == 0:
                s = jax.lax.dot_general(
                    e, ones_col, (((1,), (0,)), ((), ())),
                    preferred_element_type=jnp.float32)[:, 0:1]
            else:
                s = jnp.sum(e, axis=1, keepdims=True,
                            dtype=jnp.float32)
            whp = (wh_blk * (1.0 / (s + 1e-16))).astype(jnp.bfloat16)
            outT_ref[h] += jax.lax.dot_general(
                whp, e, (((0,), (0,)), ((), ())),
                preferred_element_type=jnp.float32)


def kernel(features, AM, W, b, att_src, att_dst):
    outT = pl.pallas_call(
        _gat_body,
        grid=(_GRID + 1,),
        in_specs=[
            pl.BlockSpec((_N, _IN_D), lambda g: (0, 0)),
            pl.BlockSpec((_BI, _N), lambda g: (jnp.maximum(g - 1, 0), 0)),
            pl.BlockSpec((_OUT_D, _IN_D), lambda g: (0, 0)),
            pl.BlockSpec((1, _OUT_D), lambda g: (0, 0)),
            pl.BlockSpec((_H, _OUT_D), lambda g: (0, 0)),
            pl.BlockSpec((_H, _OUT_D), lambda g: (0, 0)),
        ],
        out_specs=pl.BlockSpec((_H, _OUT_D, _N), lambda g: (0, 0, 0)),
        out_shape=jax.ShapeDtypeStruct((_H, _OUT_D, _N), jnp.float32),
        scratch_shapes=[
            pltpu.VMEM((_N, _OUT_D), jnp.float32),
            pltpu.VMEM((_N, _H), jnp.bfloat16),
            pltpu.VMEM((_N, _H), jnp.bfloat16),
            pltpu.VMEM((_H, _N), jnp.bfloat16),
            pltpu.VMEM((_H, _N), jnp.bfloat16),
        ],
    )(features, AM, W, jnp.reshape(b, (1, _OUT_D)), att_src, att_dst)

    return jnp.transpose(outT, (2, 0, 1)).reshape(_N, _H * _OUT_D)


# speedup vs baseline: 21136.1695x; 1.1037x over previous
import jax
import jax.numpy as jnp
from jax.experimental import pallas as pl
from jax.experimental.pallas import tpu as pltpu

_N = 2048
_IN_D = 128
_OUT_D = 16
_H = 4
_BI = 1024
_GRID = _N // _BI
_NEG_SLOPE = 0.2


def _gat_body(feat_ref, am_ref, w_ref, b_ref, asrc_ref, adst_ref,
              outT_ref,
              wh_scr, ea1_scr, ea2_scr, ed1_scr, ed2_scr):
    g = pl.program_id(0)

    @pl.when(g == 0)
    def _tables():
        wh = jax.lax.dot_general(
            feat_ref[...], w_ref[...], (((1,), (1,)), ((), ())),
            preferred_element_type=jnp.float32) + b_ref[...]
        wh_scr[...] = wh
        acs = jax.lax.dot_general(
            wh, asrc_ref[...], (((1,), (1,)), ((), ())),
            preferred_element_type=jnp.float32)
        acd = jax.lax.dot_general(
            wh, adst_ref[...], (((1,), (1,)), ((), ())),
            preferred_element_type=jnp.float32)
        acdT = jax.lax.dot_general(
            adst_ref[...], wh, (((1,), (1,)), ((), ())),
            preferred_element_type=jnp.float32)
        m = (jnp.max(acs, axis=0, keepdims=True)
             + jnp.max(acd, axis=0, keepdims=True))
        ea1_scr[...] = jnp.exp(acs - m).astype(jnp.bfloat16)
        ea2_scr[...] = jnp.exp(_NEG_SLOPE * acs - m).astype(jnp.bfloat16)
        ed1_scr[...] = jnp.exp(acdT).astype(jnp.bfloat16)
        ed2_scr[...] = jnp.exp(_NEG_SLOPE * acdT).astype(jnp.bfloat16)
        outT_ref[...] = jnp.zeros_like(outT_ref)

    @pl.when(g > 0)
    def _block():
        i = g - 1
        rows = pl.ds(i * _BI, _BI)
        maskb = am_ref[...].astype(jnp.bfloat16)
        ones_col = jnp.ones((_N, 8), dtype=jnp.bfloat16)
        ea1_blk = ea1_scr[rows, :]
        ea2_blk = ea2_scr[rows, :]
        wh_blk = wh_scr[rows, :]

        for h in range(_H):
            u = ea1_blk[:, h:h + 1] * ed1_scr[h:h + 1, :]
            v = ea2_blk[:, h:h + 1] * ed2_scr[h:h + 1, :]
            e = jnp.maximum(u, v) * maskb
            if h == 0:
                s = jax.lax.dot_general(
                    e, ones_col, (((1,), (0,)), ((), ())),
                    preferred_element_type=jnp.float32)[:, 0:1]
            else:
                eh = e[:, :_N // 2] + e[:, _N // 2:]
                eq = eh[:, :_N // 4] + eh[:, _N // 4:]
                eo = eq[:, :_N // 8] + eq[:, _N // 8:]
                s = jnp.sum(eo, axis=1, keepdims=True,
                            dtype=jnp.float32)
            whp = (wh_blk * (1.0 / (s + 1e-16))).astype(jnp.bfloat16)
            outT_ref[h] += jax.lax.dot_general(
                whp, e, (((0,), (0,)), ((), ())),
                preferred_element_type=jnp.float32)


def kernel(features, AM, W, b, att_src, att_dst):
    outT = pl.pallas_call(
        _gat_body,
        grid=(_GRID + 1,),
        in_specs=[
            pl.BlockSpec((_N, _IN_D), lambda g: (0, 0)),
            pl.BlockSpec((_BI, _N), lambda g: (jnp.maximum(g - 1, 0), 0)),
            pl.BlockSpec((_OUT_D, _IN_D), lambda g: (0, 0)),
            pl.BlockSpec((1, _OUT_D), lambda g: (0, 0)),
            pl.BlockSpec((_H, _OUT_D), lambda g: (0, 0)),
            pl.BlockSpec((_H, _OUT_D), lambda g: (0, 0)),
        ],
        out_specs=pl.BlockSpec((_H, _OUT_D, _N), lambda g: (0, 0, 0)),
        out_shape=jax.ShapeDtypeStruct((_H, _OUT_D, _N), jnp.float32),
        scratch_shapes=[
            pltpu.VMEM((_N, _OUT_D), jnp.float32),
            pltpu.VMEM((_N, _H), jnp.bfloat16),
            pltpu.VMEM((_N, _H), jnp.bfloat16),
            pltpu.VMEM((_H, _N), jnp.bfloat16),
            pltpu.VMEM((_H, _N), jnp.bfloat16),
        ],
    )(features, AM, W, jnp.reshape(b, (1, _OUT_D)), att_src, att_dst)

    return jnp.transpose(outT, (2, 0, 1)).reshape(_N, _H * _OUT_D)
